# baseline (speedup 1.0000x reference)
import functools

import jax
import jax.numpy as jnp
from jax import lax
from jax.experimental import pallas as pl
from jax.experimental.pallas import tpu as pltpu
from jax.experimental.pallas import tpu_sc as plsc

B = 4096
D = 64


@functools.lru_cache(maxsize=None)
def _build_sc_gather():
    info = plsc.get_sparse_core_info()
    nc, ns = info.num_cores, info.num_subcores
    nw = nc * ns
    b_per_w = B // nw

    mesh = plsc.VectorSubcoreMesh(core_axis_name="c", subcore_axis_name="s")

    @functools.partial(
        pl.kernel,
        mesh=mesh,
        compiler_params=pltpu.CompilerParams(use_tc_tiling_on_sc=False),
        out_type=[
            jax.ShapeDtypeStruct((B, D), jnp.float32),
            jax.ShapeDtypeStruct((B, D), jnp.float32),
        ],
        scratch_types=[
            pltpu.VMEM((b_per_w,), jnp.int32),
            pltpu.VMEM((b_per_w,), jnp.int32),
            pltpu.VMEM((b_per_w, D), jnp.float32),
            pltpu.VMEM((b_per_w, D), jnp.float32),
            pltpu.SemaphoreType.DMA,
            pltpu.SemaphoreType.DMA,
        ],
    )
    def gather_kernel(uidx_hbm, iidx_hbm, utab_hbm, itab_hbm,
                      urows_out, irows_out,
                      uidx_v, iidx_v, urows_v, irows_v, sem_u, sem_i):
        wid = lax.axis_index("s") * nc + lax.axis_index("c")
        base = wid * b_per_w
        pltpu.sync_copy(uidx_hbm.at[pl.ds(base, b_per_w)], uidx_v)
        pltpu.sync_copy(iidx_hbm.at[pl.ds(base, b_per_w)], iidx_v)
        cu = pltpu.async_copy(utab_hbm.at[uidx_v], urows_v, sem_u)
        ci = pltpu.async_copy(itab_hbm.at[iidx_v], irows_v, sem_i)
        cu.wait()
        ci.wait()
        pltpu.sync_copy(urows_v, urows_out.at[pl.ds(base, b_per_w)])
        pltpu.sync_copy(irows_v, irows_out.at[pl.ds(base, b_per_w)])

    return gather_kernel


def _outer_sigmoid_body(u_ref, v_ref, o_ref):
    u = u_ref[...]
    v = v_ref[...]
    x = u[:, :, None] * v[:, None, :]
    o_ref[...] = 0.5 * jnp.tanh(0.5 * x) + 0.5


def kernel(inputs, user_table, item_table):
    u_idx = inputs[:, 0]
    i_idx = inputs[:, 1]

    u_rows, i_rows = _build_sc_gather()(u_idx, i_idx, user_table, item_table)

    bu = 256
    out = pl.pallas_call(
        _outer_sigmoid_body,
        grid=(B // bu,),
        in_specs=[
            pl.BlockSpec((bu, D), lambda i: (i, 0)),
            pl.BlockSpec((bu, D), lambda i: (i, 0)),
        ],
        out_specs=pl.BlockSpec((bu, D, D), lambda i: (i, 0, 0)),
        out_shape=jax.ShapeDtypeStruct((B, D, D), jnp.float32),
    )(u_rows, i_rows)
    return out


# speedup vs baseline: 1.4401x; 1.4401x over previous
import functools

import jax
import jax.numpy as jnp
from jax import lax
from jax.experimental import pallas as pl
from jax.experimental.pallas import tpu as pltpu
from jax.experimental.pallas import tpu_sc as plsc

B = 4096
D = 64
V = 100000
_PACK_C = 1024


_NBLK = -(-V // _PACK_C)
_VPAD = _NBLK * (_PACK_C // 2)


def _pack_body(t_ref, o_ref):
    x = t_ref[...]
    xt = x.T
    h = _PACK_C // 2
    o_ref[...] = jnp.concatenate([xt[:h], xt[h:]], axis=1)


def _pack(table_t):
    return pl.pallas_call(
        _pack_body,
        grid=(_NBLK,),
        in_specs=[pl.BlockSpec((D, _PACK_C), lambda k: (0, k))],
        out_specs=pl.BlockSpec((_PACK_C // 2, 2 * D), lambda k: (k, 0)),
        out_shape=jax.ShapeDtypeStruct((_VPAD, 2 * D), jnp.float32),
    )(table_t)


def _row_half(idx):
    half = _PACK_C // 2
    row = (idx // _PACK_C) * half + (idx % half)
    return row, ((idx % _PACK_C) // half).astype(jnp.float32)


@functools.lru_cache(maxsize=None)
def _build_sc_gather():
    info = plsc.get_sparse_core_info()
    nc, ns = info.num_cores, info.num_subcores
    nw = nc * ns
    b_per_w = B // nw

    mesh = plsc.VectorSubcoreMesh(core_axis_name="c", subcore_axis_name="s")

    @functools.partial(
        pl.kernel,
        mesh=mesh,
        out_type=[
            jax.ShapeDtypeStruct((B, 2 * D), jnp.float32),
            jax.ShapeDtypeStruct((B, 2 * D), jnp.float32),
        ],
        scratch_types=[
            pltpu.VMEM((b_per_w,), jnp.int32),
            pltpu.VMEM((b_per_w,), jnp.int32),
            pltpu.VMEM((b_per_w, 2 * D), jnp.float32),
            pltpu.VMEM((b_per_w, 2 * D), jnp.float32),
            pltpu.SemaphoreType.DMA,
            pltpu.SemaphoreType.DMA,
        ],
    )
    def gather_kernel(uidx_hbm, iidx_hbm, utab_hbm, itab_hbm,
                      urows_out, irows_out,
                      uidx_v, iidx_v, urows_v, irows_v, sem_u, sem_i):
        wid = lax.axis_index("s") * nc + lax.axis_index("c")
        base = wid * b_per_w
        pltpu.sync_copy(uidx_hbm.at[pl.ds(base, b_per_w)], uidx_v)
        pltpu.sync_copy(iidx_hbm.at[pl.ds(base, b_per_w)], iidx_v)
        cu = pltpu.async_copy(utab_hbm.at[uidx_v], urows_v, sem_u)
        ci = pltpu.async_copy(itab_hbm.at[iidx_v], irows_v, sem_i)
        cu.wait()
        ci.wait()
        pltpu.sync_copy(urows_v, urows_out.at[pl.ds(base, b_per_w)])
        pltpu.sync_copy(irows_v, irows_out.at[pl.ds(base, b_per_w)])

    return gather_kernel


def _dense_body(ua_ref, ub_ref, va_ref, vb_ref, pu_ref, pv_ref, o_ref):
    pu = pu_ref[...]
    pv = pv_ref[...]
    ut = ua_ref[...] * (1.0 - pu) + ub_ref[...] * pu
    vt = va_ref[...] * (1.0 - pv) + vb_ref[...] * pv
    x = ut[:, None, :] * vt[None, :, :]
    o_ref[...] = 0.5 * jnp.tanh(0.5 * x) + 0.5


def kernel(inputs, user_table, item_table):
    u_idx = inputs[:, 0]
    i_idx = inputs[:, 1]
    u_row, pu = _row_half(u_idx)
    i_row, pv = _row_half(i_idx)
    pu = pu.reshape(1, B)
    pv = pv.reshape(1, B)

    utab2 = _pack(user_table.T)
    itab2 = _pack(item_table.T)
    u_wide, i_wide = _build_sc_gather()(u_row, i_row, utab2, itab2)
    uw_t = u_wide.T
    vw_t = i_wide.T

    bi = 8
    out_t = pl.pallas_call(
        _dense_body,
        grid=(D // bi,),
        in_specs=[
            pl.BlockSpec((bi, B), lambda k: (k, 0)),
            pl.BlockSpec((bi, B), lambda k: (k + D // bi, 0)),
            pl.BlockSpec((D, B), lambda k: (0, 0)),
            pl.BlockSpec((D, B), lambda k: (1, 0)),
            pl.BlockSpec((1, B), lambda k: (0, 0)),
            pl.BlockSpec((1, B), lambda k: (0, 0)),
        ],
        out_specs=pl.BlockSpec((bi, D, B), lambda k: (k, 0, 0)),
        out_shape=jax.ShapeDtypeStruct((D, D, B), jnp.float32),
    )(uw_t, uw_t, vw_t, vw_t, pu, pv)
    return jnp.transpose(out_t, (2, 0, 1))


# speedup vs baseline: 2.0474x; 1.4217x over previous
import functools

import jax
import jax.numpy as jnp
from jax import lax
from jax.experimental import pallas as pl
from jax.experimental.pallas import tpu as pltpu
from jax.experimental.pallas import tpu_sc as plsc

B = 4096
D = 64
V = 100000
_PACK_C = 2048


_NBLK = -(-V // _PACK_C)
_VPAD = _NBLK * (_PACK_C // 2)


def _pack_body(t_ref, eye_ref, o_ref):
    x = t_ref[...]
    h = _PACK_C // 2
    col = pl.program_id(0) * _PACK_C + lax.broadcasted_iota(
        jnp.int32, (D, _PACK_C), 1)
    x = jnp.where(col < V, x, 0.0)
    xs = jnp.concatenate([x[:, :h], x[:, h:]], axis=0)
    o_ref[...] = lax.dot_general(xs, eye_ref[...], (((0,), (0,)), ((), ())))


def _pack(table_t, eye):
    return pl.pallas_call(
        _pack_body,
        grid=(_NBLK,),
        in_specs=[
            pl.BlockSpec((D, _PACK_C), lambda k: (0, k)),
            pl.BlockSpec((2 * D, 2 * D), lambda k: (0, 0)),
        ],
        out_specs=pl.BlockSpec((_PACK_C // 2, 2 * D), lambda k: (k, 0)),
        out_shape=jax.ShapeDtypeStruct((_VPAD, 2 * D), jnp.float32),
    )(table_t, eye)


def _row_half(idx):
    half = _PACK_C // 2
    row = (idx // _PACK_C) * half + (idx % half)
    return row, ((idx % _PACK_C) // half).astype(jnp.float32)


@functools.lru_cache(maxsize=None)
def _build_sc_gather():
    info = plsc.get_sparse_core_info()
    nc, ns = info.num_cores, info.num_subcores
    nw = nc * ns
    b_per_w = B // nw

    mesh = plsc.VectorSubcoreMesh(core_axis_name="c", subcore_axis_name="s")

    @functools.partial(
        pl.kernel,
        mesh=mesh,
        out_type=[
            jax.ShapeDtypeStruct((B, 2 * D), jnp.float32),
            jax.ShapeDtypeStruct((B, 2 * D), jnp.float32),
        ],
        scratch_types=[
            pltpu.VMEM((b_per_w,), jnp.int32),
            pltpu.VMEM((b_per_w,), jnp.int32),
            pltpu.VMEM((b_per_w, 2 * D), jnp.float32),
            pltpu.VMEM((b_per_w, 2 * D), jnp.float32),
            pltpu.SemaphoreType.DMA,
            pltpu.SemaphoreType.DMA,
        ],
    )
    def gather_kernel(uidx_hbm, iidx_hbm, utab_hbm, itab_hbm,
                      urows_out, irows_out,
                      uidx_v, iidx_v, urows_v, irows_v, sem_u, sem_i):
        wid = lax.axis_index("s") * nc + lax.axis_index("c")
        base = wid * b_per_w
        pltpu.sync_copy(uidx_hbm.at[pl.ds(base, b_per_w)], uidx_v)
        pltpu.sync_copy(iidx_hbm.at[pl.ds(base, b_per_w)], iidx_v)
        cu = pltpu.async_copy(utab_hbm.at[uidx_v], urows_v, sem_u)
        ci = pltpu.async_copy(itab_hbm.at[iidx_v], irows_v, sem_i)
        cu.wait()
        ci.wait()
        pltpu.sync_copy(urows_v, urows_out.at[pl.ds(base, b_per_w)])
        pltpu.sync_copy(irows_v, irows_out.at[pl.ds(base, b_per_w)])

    return gather_kernel


def _dense_body(ua_ref, ub_ref, va_ref, vb_ref, pu_ref, pv_ref, o_ref):
    pu = pu_ref[...]
    pv = pv_ref[...]
    ut = ua_ref[...] * (1.0 - pu) + ub_ref[...] * pu
    vt = va_ref[...] * (1.0 - pv) + vb_ref[...] * pv
    x = ut[:, None, :] * vt[None, :, :]
    o_ref[...] = 0.5 * jnp.tanh(0.5 * x) + 0.5


def kernel(inputs, user_table, item_table):
    u_idx = inputs[:, 0]
    i_idx = inputs[:, 1]
    u_row, pu = _row_half(u_idx)
    i_row, pv = _row_half(i_idx)
    pu = pu.reshape(1, B)
    pv = pv.reshape(1, B)

    eye = jnp.eye(2 * D, dtype=jnp.float32)
    utab2 = _pack(user_table.T, eye)
    itab2 = _pack(item_table.T, eye)
    u_wide, i_wide = _build_sc_gather()(u_row, i_row, utab2, itab2)
    uw_t = u_wide.T
    vw_t = i_wide.T

    bi = 8
    out_t = pl.pallas_call(
        _dense_body,
        grid=(D // bi,),
        in_specs=[
            pl.BlockSpec((bi, B), lambda k: (k, 0)),
            pl.BlockSpec((bi, B), lambda k: (k + D // bi, 0)),
            pl.BlockSpec((D, B), lambda k: (0, 0)),
            pl.BlockSpec((D, B), lambda k: (1, 0)),
            pl.BlockSpec((1, B), lambda k: (0, 0)),
            pl.BlockSpec((1, B), lambda k: (0, 0)),
        ],
        out_specs=pl.BlockSpec((bi, D, B), lambda k: (k, 0, 0)),
        out_shape=jax.ShapeDtypeStruct((D, D, B), jnp.float32),
    )(uw_t, uw_t, vw_t, vw_t, pu, pv)
    return jnp.transpose(out_t, (2, 0, 1))


# speedup vs baseline: 2.8988x; 1.4159x over previous
import functools

import jax
import jax.numpy as jnp
from jax import lax
from jax.experimental import pallas as pl
from jax.experimental.pallas import tpu as pltpu
from jax.experimental.pallas import tpu_sc as plsc

B = 4096
D = 64
V = 100000
_PACK_C = 8192


_NBLK = -(-V // _PACK_C)
_VPAD = _NBLK * (_PACK_C // 2)


def _pack_body(t_ref, eye_ref, o_ref):
    x = t_ref[...]
    h = _PACK_C // 2
    col = pl.program_id(0) * _PACK_C + lax.broadcasted_iota(
        jnp.int32, (D, _PACK_C), 1)
    x = jnp.where(col < V, x, 0.0)
    xs = jnp.concatenate([x[:, :h], x[:, h:]], axis=0)
    o_ref[...] = lax.dot_general(xs, eye_ref[...], (((0,), (0,)), ((), ())))


def _pack(table_t, eye):
    return pl.pallas_call(
        _pack_body,
        grid=(_NBLK,),
        in_specs=[
            pl.BlockSpec((D, _PACK_C), lambda k: (0, k)),
            pl.BlockSpec((2 * D, 2 * D), lambda k: (0, 0)),
        ],
        out_specs=pl.BlockSpec((_PACK_C // 2, 2 * D), lambda k: (k, 0)),
        out_shape=jax.ShapeDtypeStruct((_VPAD, 2 * D), jnp.float32),
    )(table_t, eye)


def _row_half(idx):
    half = _PACK_C // 2
    row = (idx // _PACK_C) * half + (idx % half)
    return row, ((idx % _PACK_C) // half).astype(jnp.float32)


@functools.lru_cache(maxsize=None)
def _build_sc_gather():
    info = plsc.get_sparse_core_info()
    nc, ns = info.num_cores, info.num_subcores
    nw = nc * ns
    b_per_w = B // nw

    mesh = plsc.VectorSubcoreMesh(core_axis_name="c", subcore_axis_name="s")

    @functools.partial(
        pl.kernel,
        mesh=mesh,
        out_type=[
            jax.ShapeDtypeStruct((B, 2 * D), jnp.float32),
            jax.ShapeDtypeStruct((B, 2 * D), jnp.float32),
        ],
        scratch_types=[
            pltpu.VMEM((b_per_w,), jnp.int32),
            pltpu.VMEM((b_per_w,), jnp.int32),
            pltpu.VMEM((b_per_w, 2 * D), jnp.float32),
            pltpu.VMEM((b_per_w, 2 * D), jnp.float32),
            pltpu.SemaphoreType.DMA,
            pltpu.SemaphoreType.DMA,
        ],
    )
    def gather_kernel(uidx_hbm, iidx_hbm, utab_hbm, itab_hbm,
                      urows_out, irows_out,
                      uidx_v, iidx_v, urows_v, irows_v, sem_u, sem_i):
        wid = lax.axis_index("s") * nc + lax.axis_index("c")
        base = wid * b_per_w
        pltpu.sync_copy(uidx_hbm.at[pl.ds(base, b_per_w)], uidx_v)
        pltpu.sync_copy(iidx_hbm.at[pl.ds(base, b_per_w)], iidx_v)
        cu = pltpu.async_copy(utab_hbm.at[uidx_v], urows_v, sem_u)
        ci = pltpu.async_copy(itab_hbm.at[iidx_v], irows_v, sem_i)
        cu.wait()
        ci.wait()
        pltpu.sync_copy(urows_v, urows_out.at[pl.ds(base, b_per_w)])
        pltpu.sync_copy(irows_v, irows_out.at[pl.ds(base, b_per_w)])

    return gather_kernel


def _dense_body(ua_ref, ub_ref, va_ref, vb_ref, pu_ref, pv_ref, o_ref):
    pu = pu_ref[...]
    pv = pv_ref[...]
    ut = ua_ref[...] * (1.0 - pu) + ub_ref[...] * pu
    vt = va_ref[...] * (1.0 - pv) + vb_ref[...] * pv
    x = ut[:, None, :] * vt[None, :, :]
    o_ref[...] = 0.5 * jnp.tanh(0.5 * x) + 0.5


def kernel(inputs, user_table, item_table):
    u_idx = inputs[:, 0]
    i_idx = inputs[:, 1]
    u_row, pu = _row_half(u_idx)
    i_row, pv = _row_half(i_idx)
    pu = pu.reshape(1, B)
    pv = pv.reshape(1, B)

    eye = jnp.eye(2 * D, dtype=jnp.float32)
    utab2 = _pack(user_table.T, eye)
    itab2 = _pack(item_table.T, eye)
    u_wide, i_wide = _build_sc_gather()(u_row, i_row, utab2, itab2)
    uw_t = u_wide.T
    vw_t = i_wide.T

    bi = 8
    out_t = pl.pallas_call(
        _dense_body,
        grid=(D // bi,),
        in_specs=[
            pl.BlockSpec((bi, B), lambda k: (k, 0)),
            pl.BlockSpec((bi, B), lambda k: (k + D // bi, 0)),
            pl.BlockSpec((D, B), lambda k: (0, 0)),
            pl.BlockSpec((D, B), lambda k: (1, 0)),
            pl.BlockSpec((1, B), lambda k: (0, 0)),
            pl.BlockSpec((1, B), lambda k: (0, 0)),
        ],
        out_specs=pl.BlockSpec((bi, D, B), lambda k: (k, 0, 0)),
        out_shape=jax.ShapeDtypeStruct((D, D, B), jnp.float32),
    )(uw_t, uw_t, vw_t, vw_t, pu, pv)
    return jnp.transpose(out_t, (2, 0, 1))


# speedup vs baseline: 2.9508x; 1.0179x over previous
import functools

import jax
import jax.numpy as jnp
from jax import lax
from jax.experimental import pallas as pl
from jax.experimental.pallas import tpu as pltpu
from jax.experimental.pallas import tpu_sc as plsc

B = 4096
D = 64
V = 100000
_PACK_C = 8192


_NBLK = -(-V // _PACK_C)
_VPAD = _NBLK * (_PACK_C // 2)


def _pack_body(t_ref, eye_ref, o_ref):
    x = t_ref[...]
    h = _PACK_C // 2
    col = pl.program_id(0) * _PACK_C + lax.broadcasted_iota(
        jnp.int32, (D, _PACK_C), 1)
    x = jnp.where(col < V, x, 0.0)
    xs = jnp.concatenate([x[:, :h], x[:, h:]], axis=0)
    o_ref[...] = lax.dot_general(xs, eye_ref[...], (((0,), (0,)), ((), ())))


def _pack(table_t, eye):
    return pl.pallas_call(
        _pack_body,
        grid=(_NBLK,),
        in_specs=[
            pl.BlockSpec((D, _PACK_C), lambda k: (0, k)),
            pl.BlockSpec((2 * D, 2 * D), lambda k: (0, 0)),
        ],
        out_specs=pl.BlockSpec((_PACK_C // 2, 2 * D), lambda k: (k, 0)),
        out_shape=jax.ShapeDtypeStruct((_VPAD, 2 * D), jnp.float32),
    )(table_t, eye)


def _row_half(idx):
    half = _PACK_C // 2
    row = (idx // _PACK_C) * half + (idx % half)
    return row, ((idx % _PACK_C) // half).astype(jnp.float32)


@functools.lru_cache(maxsize=None)
def _build_sc_gather():
    info = plsc.get_sparse_core_info()
    nc, ns = info.num_cores, info.num_subcores
    nw = nc * ns
    b_per_w = B // nw

    mesh = plsc.VectorSubcoreMesh(core_axis_name="c", subcore_axis_name="s")

    @functools.partial(
        pl.kernel,
        mesh=mesh,
        out_type=jax.ShapeDtypeStruct((B, 2 * D), jnp.float32),
        scratch_types=[
            pltpu.VMEM((b_per_w,), jnp.int32),
            pltpu.VMEM((b_per_w, 2 * D), jnp.float32),
            pltpu.SemaphoreType.DMA,
        ],
    )
    def gather_kernel(idx_hbm, tab_hbm, rows_out, idx_v, rows_v, sem):
        wid = lax.axis_index("s") * nc + lax.axis_index("c")
        base = wid * b_per_w
        pltpu.sync_copy(idx_hbm.at[pl.ds(base, b_per_w)], idx_v)
        pltpu.async_copy(tab_hbm.at[idx_v], rows_v, sem).wait()
        pltpu.sync_copy(rows_v, rows_out.at[pl.ds(base, b_per_w)])

    return gather_kernel


def _dense_body(ua_ref, ub_ref, va_ref, vb_ref, pu_ref, pv_ref, o_ref):
    pu = pu_ref[...]
    pv = pv_ref[...]
    ut = ua_ref[...] * (1.0 - pu) + ub_ref[...] * pu
    vt = va_ref[...] * (1.0 - pv) + vb_ref[...] * pv
    x = ut[:, None, :] * vt[None, :, :]
    o_ref[...] = 0.5 * jnp.tanh(0.5 * x) + 0.5


def kernel(inputs, user_table, item_table):
    u_idx = inputs[:, 0]
    i_idx = inputs[:, 1]
    u_row, pu = _row_half(u_idx)
    i_row, pv = _row_half(i_idx)
    pu = pu.reshape(1, B)
    pv = pv.reshape(1, B)

    eye = jnp.eye(2 * D, dtype=jnp.float32)
    gather = _build_sc_gather()
    utab2 = _pack(user_table.T, eye)
    u_wide = gather(u_row, utab2)
    itab2 = _pack(item_table.T, eye)
    uw_t = u_wide.T
    i_wide = gather(i_row, itab2)
    vw_t = i_wide.T

    bi = 8
    out_t = pl.pallas_call(
        _dense_body,
        grid=(D // bi,),
        in_specs=[
            pl.BlockSpec((bi, B), lambda k: (k, 0)),
            pl.BlockSpec((bi, B), lambda k: (k + D // bi, 0)),
            pl.BlockSpec((D, B), lambda k: (0, 0)),
            pl.BlockSpec((D, B), lambda k: (1, 0)),
            pl.BlockSpec((1, B), lambda k: (0, 0)),
            pl.BlockSpec((1, B), lambda k: (0, 0)),
        ],
        out_specs=pl.BlockSpec((bi, D, B), lambda k: (k, 0, 0)),
        out_shape=jax.ShapeDtypeStruct((D, D, B), jnp.float32),
    )(uw_t, uw_t, vw_t, vw_t, pu, pv)
    return jnp.transpose(out_t, (2, 0, 1))
